# baseline (speedup 1.0000x reference)
import jax
import jax.numpy as jnp
from jax import lax
from jax.experimental import pallas as pl
from jax.experimental.pallas import tpu as pltpu
from jax.experimental.pallas import tpu_sc as plsc

S = 2048
D = 1024
NW = 32
RPW = (2 * S) // NW
CR = 32
NCH = RPW // CR
L = 16


def _sc_body(bos_hbm, idx_hbm, embeds_hbm, speech_hbm, pos_hbm, out_hbm,
             buf0, buf1, tokbuf, ixbuf, row_s, row_p,
             sem0, sem1, gsem):
    wid = lax.axis_index("s") * 2 + lax.axis_index("c")
    b = wid // 16
    r0 = (wid % 16) * RPW
    bufs = (buf0, buf1)
    sems = (sem0, sem1)

    def rd(c, buf, sem):
        return pltpu.async_copy(
            embeds_hbm.at[b, pl.ds(r0 + c * CR, CR), :], buf, sem)

    def wr(c, buf, sem):
        return pltpu.async_copy(
            buf, out_hbm.at[b, pl.ds(r0 + c * CR, CR), :], sem)

    reads = [rd(0, buf0, sem0), rd(1, buf1, sem1)]
    writes = [None] * NCH
    for c in range(NCH):
        k = c % 2
        if c >= 2:
            writes[c - 2].wait()
            reads.append(rd(c, bufs[k], sems[k]))
        reads[c].wait()
        writes[c] = wr(c, bufs[k], sems[k])

    @pl.when(wid == 0)
    def _bos():
        pltpu.sync_copy(bos_hbm, tokbuf)
        pltpu.sync_copy(idx_hbm, ixbuf)
        pltpu.async_copy(speech_hbm.at[tokbuf], row_s, gsem).wait()
        pltpu.async_copy(pos_hbm.at[ixbuf], row_p, gsem).wait()
        for i in range(D // L):
            sl = pl.ds(i * L, L)
            row_s[0, sl] = row_s[0, sl] + row_p[0, sl]
        pltpu.sync_copy(row_s, out_hbm.at[0, pl.ds(S, 1), :])
        pltpu.sync_copy(row_s, out_hbm.at[1, pl.ds(S, 1), :])

    writes[NCH - 2].wait()
    writes[NCH - 1].wait()


def kernel(bos_token, embeds, idx, speech_emb, pos_emb):
    mesh = plsc.VectorSubcoreMesh(core_axis_name="c", subcore_axis_name="s")
    sc_call = pl.kernel(
        _sc_body,
        mesh=mesh,
        out_type=jax.ShapeDtypeStruct((2, S + 1, D), jnp.float32),
        scratch_types=[
            pltpu.VMEM((CR, D), jnp.float32),
            pltpu.VMEM((CR, D), jnp.float32),
            pltpu.VMEM((1,), jnp.int32),
            pltpu.VMEM((1,), jnp.int32),
            pltpu.VMEM((1, D), jnp.float32),
            pltpu.VMEM((1, D), jnp.float32),
            pltpu.SemaphoreType.DMA,
            pltpu.SemaphoreType.DMA,
            pltpu.SemaphoreType.DMA,
        ],
    )
    return sc_call(bos_token.reshape(1), idx, embeds, speech_emb, pos_emb)


# speedup vs baseline: 2.2596x; 2.2596x over previous
import jax
import jax.numpy as jnp
from jax import lax
from jax.experimental import pallas as pl
from jax.experimental.pallas import tpu as pltpu
from jax.experimental.pallas import tpu_sc as plsc

S = 2048
D = 1024
NW = 32
RPW = S // NW
CRK = 16
NCH = RPW // CRK
L = 16


def _sc_body(bos_hbm, idx_hbm, embeds_hbm, speech_hbm, pos_hbm, out_hbm,
             buf0, buf1, tokbuf, ixbuf, row_s, row_p, row_i,
             rsem, wsem, gsem):
    wid = lax.axis_index("s") * 2 + lax.axis_index("c")
    r0 = wid * RPW
    bufs = (buf0, buf1)

    def rd(c, k, b):
        return pltpu.async_copy(
            embeds_hbm.at[b, pl.ds(r0 + c * CRK, CRK), :],
            bufs[k].at[:, b, :], rsem.at[k, b])

    def wr(c, k):
        return pltpu.async_copy(
            bufs[k], out_hbm.at[pl.ds(r0 + c * CRK, CRK), :, :], wsem.at[k])

    reads = {}
    for c in (0, 1):
        for b in (0, 1):
            reads[(c, b)] = rd(c, c, b)
    writes = [None] * NCH
    for c in range(NCH):
        k = c % 2
        if c >= 2:
            writes[c - 2].wait()
            for b in (0, 1):
                reads[(c, b)] = rd(c, k, b)
        reads[(c, 0)].wait()
        reads[(c, 1)].wait()
        writes[c] = wr(c, k)

    @pl.when(wid == 0)
    def _bos():
        pltpu.sync_copy(bos_hbm, tokbuf)
        pltpu.sync_copy(idx_hbm, ixbuf)
        pltpu.async_copy(speech_hbm.at[tokbuf], row_s, gsem).wait()
        pltpu.async_copy(pos_hbm.at[ixbuf], row_p, gsem).wait()
        for i in range(D // L):
            sl = pl.ds(i * L, L)
            v = row_s[0, sl] + row_p[0, sl]
            row_i[0, 0, sl] = v
            row_i[0, 1, sl] = v
        pltpu.sync_copy(row_i, out_hbm.at[pl.ds(S, 1), :, :])

    writes[NCH - 2].wait()
    writes[NCH - 1].wait()


def kernel(bos_token, embeds, idx, speech_emb, pos_emb):
    mesh = plsc.VectorSubcoreMesh(core_axis_name="c", subcore_axis_name="s")
    sc_call = pl.kernel(
        _sc_body,
        mesh=mesh,
        out_type=jax.ShapeDtypeStruct((S + 1, 2, D), jnp.float32),
        scratch_types=[
            pltpu.VMEM((CRK, 2, D), jnp.float32),
            pltpu.VMEM((CRK, 2, D), jnp.float32),
            pltpu.VMEM((1,), jnp.int32),
            pltpu.VMEM((1,), jnp.int32),
            pltpu.VMEM((1, D), jnp.float32),
            pltpu.VMEM((1, D), jnp.float32),
            pltpu.VMEM((1, 2, D), jnp.float32),
            pltpu.SemaphoreType.DMA((2, 2)),
            pltpu.SemaphoreType.DMA((2,)),
            pltpu.SemaphoreType.DMA,
        ],
    )
    out_t = sc_call(bos_token.reshape(1), idx, embeds, speech_emb, pos_emb)
    return jnp.transpose(out_t, (1, 0, 2))


# speedup vs baseline: 2.3428x; 1.0368x over previous
import jax
import jax.numpy as jnp
from jax import lax
from jax.experimental import pallas as pl
from jax.experimental.pallas import tpu as pltpu
from jax.experimental.pallas import tpu_sc as plsc

S = 2048
D = 1024
NW = 32
RPW = S // NW
CRK = 16
NCH = RPW // CRK
L = 16


def _sc_body(bos_hbm, idx_hbm, embeds_hbm, speech_hbm, pos_hbm, out_hbm,
             buf0, buf1, tokbuf, ixbuf, row_s, row_p, row_i,
             rsem, wsem, gsem_t, gsem_i, gsem_s, gsem_p):
    cid = lax.axis_index("c")
    sid = lax.axis_index("s")
    wid = sid * 2 + cid
    r0 = wid * RPW
    bufs = (buf0, buf1)
    is_bos = sid == 0

    @pl.when(is_bos)
    def _bos_fetch():
        pltpu.async_copy(bos_hbm, tokbuf, gsem_t)
        pltpu.async_copy(idx_hbm, ixbuf, gsem_i)

    def rd(c, k, b):
        return pltpu.async_copy(
            embeds_hbm.at[b, pl.ds(r0 + c * CRK, CRK), :],
            bufs[k].at[:, b, :], rsem.at[k, b])

    def wr(c, k):
        return pltpu.async_copy(
            bufs[k], out_hbm.at[pl.ds(r0 + c * CRK, CRK), :, :], wsem.at[k])

    reads = {}
    for c in (0, 1):
        for b in (0, 1):
            reads[(c, b)] = rd(c, c, b)

    @pl.when(is_bos)
    def _bos_gather():
        pltpu.make_async_copy(bos_hbm, tokbuf, gsem_t).wait()
        pltpu.make_async_copy(idx_hbm, ixbuf, gsem_i).wait()
        pltpu.async_copy(speech_hbm.at[tokbuf], row_s, gsem_s)
        pltpu.async_copy(pos_hbm.at[ixbuf], row_p, gsem_p)

    writes = [None] * NCH
    for c in range(NCH):
        k = c % 2
        if c >= 2:
            writes[c - 2].wait()
            for b in (0, 1):
                reads[(c, b)] = rd(c, k, b)
        reads[(c, 0)].wait()
        reads[(c, 1)].wait()
        writes[c] = wr(c, k)

    @pl.when(is_bos)
    def _bos_write():
        pltpu.make_async_copy(speech_hbm.at[tokbuf], row_s, gsem_s).wait()
        pltpu.make_async_copy(pos_hbm.at[ixbuf], row_p, gsem_p).wait()
        for i in range(D // L):
            sl = pl.ds(i * L, L)
            row_i[0, 0, sl] = row_s[0, sl] + row_p[0, sl]
        pltpu.sync_copy(
            row_i, out_hbm.at[pl.ds(S, 1), pl.ds(cid, 1), :])

    writes[NCH - 2].wait()
    writes[NCH - 1].wait()


def kernel(bos_token, embeds, idx, speech_emb, pos_emb):
    mesh = plsc.VectorSubcoreMesh(core_axis_name="c", subcore_axis_name="s")
    sc_call = pl.kernel(
        _sc_body,
        mesh=mesh,
        out_type=jax.ShapeDtypeStruct((S + 1, 2, D), jnp.float32),
        scratch_types=[
            pltpu.VMEM((CRK, 2, D), jnp.float32),
            pltpu.VMEM((CRK, 2, D), jnp.float32),
            pltpu.VMEM((1,), jnp.int32),
            pltpu.VMEM((1,), jnp.int32),
            pltpu.VMEM((1, D), jnp.float32),
            pltpu.VMEM((1, D), jnp.float32),
            pltpu.VMEM((1, 1, D), jnp.float32),
            pltpu.SemaphoreType.DMA((2, 2)),
            pltpu.SemaphoreType.DMA((2,)),
            pltpu.SemaphoreType.DMA,
            pltpu.SemaphoreType.DMA,
            pltpu.SemaphoreType.DMA,
            pltpu.SemaphoreType.DMA,
        ],
    )
    out_t = sc_call(bos_token.reshape(1), idx, embeds, speech_emb, pos_emb)
    return jnp.transpose(out_t, (1, 0, 2))


# speedup vs baseline: 2.3914x; 1.0208x over previous
import jax
import jax.numpy as jnp
from jax import lax
from jax.experimental import pallas as pl
from jax.experimental.pallas import tpu as pltpu
from jax.experimental.pallas import tpu_sc as plsc

S = 2048
D = 1024
NW = 32
RPW = S // NW
CRK = 16
NCH = RPW // CRK
L = 16


def _sc_body(bos_hbm, idx_hbm, embeds_hbm, speech_hbm, pos_hbm, out_hbm,
             buf0, buf1, buf2, tokbuf, ixbuf, row_s, row_p, row_i,
             rsem, wsem, gsem_t, gsem_i, gsem_s, gsem_p):
    cid = lax.axis_index("c")
    sid = lax.axis_index("s")
    wid = sid * 2 + cid
    r0 = wid * RPW
    bufs = (buf0, buf1, buf2)
    is_bos = sid == 0

    @pl.when(is_bos)
    def _bos_fetch():
        pltpu.async_copy(bos_hbm, tokbuf, gsem_t)
        pltpu.async_copy(idx_hbm, ixbuf, gsem_i)

    def rd(c, k, b):
        return pltpu.async_copy(
            embeds_hbm.at[b, pl.ds(r0 + c * CRK, CRK), :],
            bufs[k].at[:, b, :], rsem.at[k, b])

    def wr(c, k):
        return pltpu.async_copy(
            bufs[k], out_hbm.at[pl.ds(r0 + c * CRK, CRK), :, :], wsem.at[k])

    reads = {}
    for c in (0, 1, 2):
        for b in (0, 1):
            reads[(c, b)] = rd(c, c, b)

    @pl.when(is_bos)
    def _bos_gather():
        pltpu.make_async_copy(bos_hbm, tokbuf, gsem_t).wait()
        pltpu.make_async_copy(idx_hbm, ixbuf, gsem_i).wait()
        pltpu.async_copy(speech_hbm.at[tokbuf], row_s, gsem_s)
        pltpu.async_copy(pos_hbm.at[ixbuf], row_p, gsem_p)

    writes = [None] * NCH
    for c in range(NCH):
        k = c % 3
        if c >= 3:
            writes[c - 3].wait()
            for b in (0, 1):
                reads[(c, b)] = rd(c, k, b)
        reads[(c, 0)].wait()
        reads[(c, 1)].wait()
        writes[c] = wr(c, k)

    @pl.when(is_bos)
    def _bos_write():
        pltpu.make_async_copy(speech_hbm.at[tokbuf], row_s, gsem_s).wait()
        pltpu.make_async_copy(pos_hbm.at[ixbuf], row_p, gsem_p).wait()
        for i in range(D // L):
            sl = pl.ds(i * L, L)
            row_i[0, 0, sl] = row_s[0, sl] + row_p[0, sl]
        pltpu.sync_copy(
            row_i, out_hbm.at[pl.ds(S, 1), pl.ds(cid, 1), :])

    writes[NCH - 3].wait()
    writes[NCH - 2].wait()
    writes[NCH - 1].wait()


def kernel(bos_token, embeds, idx, speech_emb, pos_emb):
    mesh = plsc.VectorSubcoreMesh(core_axis_name="c", subcore_axis_name="s")
    sc_call = pl.kernel(
        _sc_body,
        mesh=mesh,
        out_type=jax.ShapeDtypeStruct((S + 1, 2, D), jnp.float32),
        scratch_types=[
            pltpu.VMEM((CRK, 2, D), jnp.float32),
            pltpu.VMEM((CRK, 2, D), jnp.float32),
            pltpu.VMEM((CRK, 2, D), jnp.float32),
            pltpu.VMEM((1,), jnp.int32),
            pltpu.VMEM((1,), jnp.int32),
            pltpu.VMEM((1, D), jnp.float32),
            pltpu.VMEM((1, D), jnp.float32),
            pltpu.VMEM((1, 1, D), jnp.float32),
            pltpu.SemaphoreType.DMA((3, 2)),
            pltpu.SemaphoreType.DMA((3,)),
            pltpu.SemaphoreType.DMA,
            pltpu.SemaphoreType.DMA,
            pltpu.SemaphoreType.DMA,
            pltpu.SemaphoreType.DMA,
        ],
    )
    out_t = sc_call(bos_token.reshape(1), idx, embeds, speech_emb, pos_emb)
    return jnp.transpose(out_t, (1, 0, 2))
